# baseline (speedup 1.0000x reference)
import functools

import jax
import jax.numpy as jnp
from jax import lax
from jax.experimental import pallas as pl
from jax.experimental.pallas import tpu as pltpu
from jax.experimental.pallas import tpu_sc as plsc

B, L, D = 1024, 50, 768
N = B * L
EPS = 1e-12
NL = 16
NJ = D // NL

NC, NS = 2, 16
NW = NC * NS
SPW = B // NW
LP = 56


def _allsum(v):
    for sh in (8, 4, 2, 1):
        idx = lax.iota(jnp.int32, NL) ^ sh
        v = v + lax.gather(
            v, idx[:, None],
            lax.GatherDimensionNumbers(
                offset_dims=(), collapsed_slice_dims=(0,),
                start_index_map=(0,)),
            slice_sizes=(1,),
            mode=lax.GatherScatterMode.PROMISE_IN_BOUNDS)
    return v


def _ln_row(rows_v, padd_v, g_v, b_v, i):
    sum_v = jnp.zeros((NL,), jnp.float32)
    sq_v = jnp.zeros((NL,), jnp.float32)
    for j in range(NJ):
        v = rows_v[i, pl.ds(j * NL, NL)] + padd_v[pl.ds(i * D + j * NL, NL)]
        rows_v[i, pl.ds(j * NL, NL)] = v
        sum_v = sum_v + v
        sq_v = sq_v + v * v
    mu = _allsum(sum_v) * (1.0 / D)
    var = _allsum(sq_v) * (1.0 / D) - mu * mu
    xr = var + EPS
    seed = jnp.full((NL,), 0x5F3759DF, dtype=jnp.int32) - (
        lax.bitcast_convert_type(xr, jnp.int32) >> 1)
    y = lax.bitcast_convert_type(seed, jnp.float32)
    for _ in range(2):
        y = y * (1.5 - 0.5 * xr * y * y)
    for j in range(NJ):
        v = rows_v[i, pl.ds(j * NL, NL)]
        g = g_v[pl.ds(j * NL, NL)]
        bta = b_v[pl.ds(j * NL, NL)]
        rows_v[i, pl.ds(j * NL, NL)] = (v - mu) * y * g + bta


def _sc_fused(ids3, table, padd, gamma, beta):
    mesh = plsc.VectorSubcoreMesh(core_axis_name="c", subcore_axis_name="s")

    @functools.partial(
        pl.kernel,
        mesh=mesh,
        out_type=jax.ShapeDtypeStruct((B, LP, D), jnp.float32),
        scratch_types=[
            pltpu.VMEM((SPW * LP,), jnp.int32),
            pltpu.VMEM((LP, D), jnp.float32),
            pltpu.VMEM((LP, D), jnp.float32),
            pltpu.VMEM((L * D,), jnp.float32),
            pltpu.VMEM((D,), jnp.float32),
            pltpu.VMEM((D,), jnp.float32),
            pltpu.SemaphoreType.DMA,
            pltpu.SemaphoreType.DMA,
            pltpu.SemaphoreType.DMA,
            pltpu.SemaphoreType.DMA,
        ],
    )
    def k(ids_hbm, table_hbm, padd_hbm, g_hbm, b_hbm, out_hbm,
          idx_v, rows_a, rows_b, padd_v, g_v, b_v, gs_a, gs_b, os_a, os_b):
        wid = lax.axis_index("s") * NC + lax.axis_index("c")
        seq0 = wid * SPW

        pltpu.sync_copy(ids_hbm.at[wid], idx_v)
        pltpu.sync_copy(padd_hbm, padd_v)
        pltpu.sync_copy(g_hbm, g_v)
        pltpu.sync_copy(b_hbm, b_v)

        rows = (rows_a, rows_b)
        gsem = (gs_a, gs_b)
        osem = (os_a, os_b)

        def gather_start(s, buf):
            pltpu.make_async_copy(
                table_hbm.at[idx_v.at[pl.ds(s * LP, LP)]], rows[buf],
                gsem[buf]).start()

        def gather_wait(buf):
            pltpu.make_async_copy(
                table_hbm.at[idx_v.at[pl.ds(0, LP)]], rows[buf],
                gsem[buf]).wait()

        def out_start(s, buf):
            pltpu.make_async_copy(
                rows[buf], out_hbm.at[seq0 + s], osem[buf]).start()

        def out_wait(buf):
            pltpu.make_async_copy(
                rows[buf], out_hbm.at[seq0], osem[buf]).wait()

        def compute(buf):
            @plsc.parallel_loop(0, L)
            def _(i):
                _ln_row(rows[buf], padd_v, g_v, b_v, i)

        gather_start(0, 0)
        gather_start(1, 1)

        def phase(s, buf, issue_next):
            gather_wait(buf)
            compute(buf)
            out_start(s, buf)
            out_wait(buf)
            if issue_next:
                gather_start(s + 2, buf)

        def loop_body(ss, c):
            for buf in range(2):
                phase(ss * 2 + buf, buf, True)
            return c

        lax.fori_loop(0, SPW // 2 - 1, loop_body, 0)
        for buf in range(2):
            phase(SPW - 2 + buf, buf, False)

    return k(ids3, table, padd, gamma, beta)


def kernel(input_ids, word_emb, pos_emb, type_emb, ln_gamma, ln_beta):
    ids32 = input_ids.astype(jnp.int32)
    ids_p = jnp.concatenate([ids32, ids32[:, :LP - L]], axis=1)
    ids3 = ids_p.reshape(NW, SPW * LP)
    padd = (pos_emb[:L] + type_emb[0][None, :]).reshape(-1)
    out_p = _sc_fused(ids3, word_emb, padd, ln_gamma, ln_beta)
    return out_p[:, :L, :]


# speedup vs baseline: 1.3556x; 1.3556x over previous
import functools

import jax
import jax.numpy as jnp
from jax import lax
from jax.experimental import pallas as pl
from jax.experimental.pallas import tpu as pltpu
from jax.experimental.pallas import tpu_sc as plsc

B, L, D = 1024, 50, 768
N = B * L
EPS = 1e-12

NC, NS = 2, 16
NW = NC * NS
SPW = B // NW
LP = 56
NBUF = 3


def _sc_gather(ids3, table):
    mesh = plsc.VectorSubcoreMesh(core_axis_name="c", subcore_axis_name="s")

    @functools.partial(
        pl.kernel,
        mesh=mesh,
        out_type=jax.ShapeDtypeStruct((B * LP, D), jnp.float32),
        scratch_types=[
            pltpu.VMEM((SPW, LP), jnp.int32),
            pltpu.VMEM((LP, D), jnp.float32),
            pltpu.VMEM((LP, D), jnp.float32),
            pltpu.SemaphoreType.DMA,
            pltpu.SemaphoreType.DMA,
            pltpu.SemaphoreType.DMA,
            pltpu.SemaphoreType.DMA,
        ],
    )
    def k(ids_hbm, table_hbm, out_hbm, idx_v, rows_a, rows_b, gs_a, gs_b,
          os_a, os_b):
        wid = lax.axis_index("s") * NC + lax.axis_index("c")
        seq0 = wid * SPW

        pltpu.sync_copy(ids_hbm.at[wid], idx_v)

        rows = (rows_a, rows_b)
        gsem = (gs_a, gs_b)
        osem = (os_a, os_b)

        def gather_start(p, buf):
            pltpu.make_async_copy(
                table_hbm.at[idx_v.at[p]],
                rows[buf], gsem[buf]).start()

        def gather_wait(buf):
            pltpu.make_async_copy(
                table_hbm.at[idx_v.at[0]],
                rows[buf], gsem[buf]).wait()

        def out_start(p, buf):
            pltpu.make_async_copy(
                rows[buf], out_hbm.at[pl.ds((seq0 + p) * LP, LP)],
                osem[buf]).start()

        def out_wait(buf):
            pltpu.make_async_copy(
                rows[buf], out_hbm.at[pl.ds(0, LP)], osem[buf]).wait()

        gather_start(0, 0)
        gather_start(1, 1)

        def loop_body(pp, c):
            for buf in range(2):
                p = pp * 2 + buf
                gather_wait(buf)
                out_start(p, buf)
                out_wait(buf)
                gather_start(p + 2, buf)
            return c

        lax.fori_loop(0, SPW // 2 - 1, loop_body, 0)
        for buf in range(2):
            p = SPW - 2 + buf
            gather_wait(buf)
            out_start(p, buf)
            out_wait(buf)

    return k(ids3, table)


SB = 16


def _ln_body(x_ref, padd_ref, g_ref, bta_ref, o_ref):
    e = x_ref[:, :L, :] + padd_ref[...][None, :, :]
    mu = jnp.mean(e, axis=-1, keepdims=True)
    d = e - mu
    var = jnp.mean(d * d, axis=-1, keepdims=True)
    o_ref[...] = d * lax.rsqrt(var + EPS) * g_ref[...][None, :, :] \
        + bta_ref[...][None, :, :]


def _tc_add_ln(stag, padd, gamma2, beta2):
    return pl.pallas_call(
        _ln_body,
        grid=(B // SB,),
        in_specs=[
            pl.BlockSpec((SB, LP, D), lambda i: (i, 0, 0)),
            pl.BlockSpec((L, D), lambda i: (0, 0)),
            pl.BlockSpec((1, D), lambda i: (0, 0)),
            pl.BlockSpec((1, D), lambda i: (0, 0)),
        ],
        out_specs=pl.BlockSpec((SB, L, D), lambda i: (i, 0, 0)),
        out_shape=jax.ShapeDtypeStruct((B, L, D), jnp.float32),
        compiler_params=pltpu.CompilerParams(
            dimension_semantics=("arbitrary",),
        ),
    )(stag, padd, gamma2, beta2)


def kernel(input_ids, word_emb, pos_emb, type_emb, ln_gamma, ln_beta):
    ids32 = input_ids.astype(jnp.int32)
    ids_p = jnp.concatenate([ids32, ids32[:, :LP - L]], axis=1)
    ids3 = ids_p.reshape(NW, SPW, LP)
    stag = _sc_gather(ids3, word_emb).reshape(B, LP, D)
    padd = pos_emb[:L] + type_emb[0][None, :]
    return _tc_add_ln(stag, padd, ln_gamma.reshape(1, D),
                      ln_beta.reshape(1, D))


# speedup vs baseline: 1.3969x; 1.0304x over previous
import functools

import jax
import jax.numpy as jnp
from jax import lax
from jax.experimental import pallas as pl
from jax.experimental.pallas import tpu as pltpu
from jax.experimental.pallas import tpu_sc as plsc

B, L, D = 1024, 50, 768
N = B * L
EPS = 1e-12

NC, NS = 2, 16
NW = NC * NS
SPW = B // NW
LP = 56
NBUF = 3


def _sc_gather(ids3, table):
    mesh = plsc.VectorSubcoreMesh(core_axis_name="c", subcore_axis_name="s")

    @functools.partial(
        pl.kernel,
        mesh=mesh,
        out_type=jax.ShapeDtypeStruct((B * LP, D), jnp.float32),
        scratch_types=[
            pltpu.VMEM((SPW, LP), jnp.int32),
            pltpu.VMEM((LP, D), jnp.float32),
            pltpu.VMEM((LP, D), jnp.float32),
            pltpu.SemaphoreType.DMA,
            pltpu.SemaphoreType.DMA,
            pltpu.SemaphoreType.DMA,
            pltpu.SemaphoreType.DMA,
        ],
    )
    def k(ids_hbm, table_hbm, out_hbm, idx_v, rows_a, rows_b, gs_a, gs_b,
          os_a, os_b):
        wid = lax.axis_index("s") * NC + lax.axis_index("c")
        seq0 = wid * SPW

        pltpu.sync_copy(ids_hbm.at[wid], idx_v)

        rows = (rows_a, rows_b)
        gsem = (gs_a, gs_b)
        osem = (os_a, os_b)

        def gather_start(p, buf):
            pltpu.make_async_copy(
                table_hbm.at[idx_v.at[p]],
                rows[buf], gsem[buf]).start()

        def gather_wait(buf):
            pltpu.make_async_copy(
                table_hbm.at[idx_v.at[0]],
                rows[buf], gsem[buf]).wait()

        def out_start(p, buf):
            pltpu.make_async_copy(
                rows[buf], out_hbm.at[pl.ds((seq0 + p) * LP, LP)],
                osem[buf]).start()

        def out_wait(buf):
            pltpu.make_async_copy(
                rows[buf], out_hbm.at[pl.ds(0, LP)], osem[buf]).wait()

        gather_start(0, 0)
        gather_start(1, 1)

        def loop_body(pp, c):
            for buf in range(2):
                p = pp * 2 + buf
                gather_wait(buf)
                out_start(p, buf)
                out_wait(buf)
                gather_start(p + 2, buf)
            return c

        lax.fori_loop(0, SPW // 2 - 1, loop_body, 0)
        for buf in range(2):
            p = SPW - 2 + buf
            gather_wait(buf)
            out_start(p, buf)
            out_wait(buf)

    return k(ids3, table)


SB = 32


def _ln_body(x_ref, padd_ref, g_ref, bta_ref, o_ref):
    e = x_ref[:, :L, :] + padd_ref[...][None, :, :]
    mu = jnp.mean(e, axis=-1, keepdims=True)
    d = e - mu
    var = jnp.mean(d * d, axis=-1, keepdims=True)
    o_ref[...] = d * lax.rsqrt(var + EPS) * g_ref[...][None, :, :] \
        + bta_ref[...][None, :, :]


def _tc_add_ln(stag, padd, gamma2, beta2):
    return pl.pallas_call(
        _ln_body,
        grid=(B // SB,),
        in_specs=[
            pl.BlockSpec((SB, LP, D), lambda i: (i, 0, 0)),
            pl.BlockSpec((L, D), lambda i: (0, 0)),
            pl.BlockSpec((1, D), lambda i: (0, 0)),
            pl.BlockSpec((1, D), lambda i: (0, 0)),
        ],
        out_specs=pl.BlockSpec((SB, L, D), lambda i: (i, 0, 0)),
        out_shape=jax.ShapeDtypeStruct((B, L, D), jnp.float32),
        compiler_params=pltpu.CompilerParams(
            dimension_semantics=("arbitrary",),
        ),
    )(stag, padd, gamma2, beta2)


def kernel(input_ids, word_emb, pos_emb, type_emb, ln_gamma, ln_beta):
    ids32 = input_ids.astype(jnp.int32)
    ids_p = jnp.concatenate([ids32, ids32[:, :LP - L]], axis=1)
    ids3 = ids_p.reshape(NW, SPW, LP)
    stag = _sc_gather(ids3, word_emb).reshape(B, LP, D)
    padd = pos_emb[:L] + type_emb[0][None, :]
    return _tc_add_ln(stag, padd, ln_gamma.reshape(1, D),
                      ln_beta.reshape(1, D))


# speedup vs baseline: 1.3972x; 1.0002x over previous
import functools

import jax
import jax.numpy as jnp
from jax import lax
from jax.experimental import pallas as pl
from jax.experimental.pallas import tpu as pltpu
from jax.experimental.pallas import tpu_sc as plsc

B, L, D = 1024, 50, 768
N = B * L
EPS = 1e-12

NC, NS = 2, 16
NW = NC * NS
SPW = B // NW
LP = 56
NBUF = 3


def _sc_gather(ids3, table):
    mesh = plsc.VectorSubcoreMesh(core_axis_name="c", subcore_axis_name="s")

    @functools.partial(
        pl.kernel,
        mesh=mesh,
        out_type=jax.ShapeDtypeStruct((B * LP, D), jnp.float32),
        scratch_types=[
            pltpu.VMEM((SPW, LP), jnp.int32),
            pltpu.VMEM((LP, D), jnp.float32),
            pltpu.VMEM((LP, D), jnp.float32),
            pltpu.SemaphoreType.DMA,
            pltpu.SemaphoreType.DMA,
            pltpu.SemaphoreType.DMA,
            pltpu.SemaphoreType.DMA,
        ],
    )
    def k(ids_hbm, table_hbm, out_hbm, idx_v, rows_a, rows_b, gs_a, gs_b,
          os_a, os_b):
        wid = lax.axis_index("s") * NC + lax.axis_index("c")
        seq0 = wid * SPW

        pltpu.sync_copy(ids_hbm.at[wid], idx_v)

        rows = (rows_a, rows_b)
        gsem = (gs_a, gs_b)
        osem = (os_a, os_b)

        def gather_start(p, buf):
            pltpu.make_async_copy(
                table_hbm.at[idx_v.at[p]],
                rows[buf], gsem[buf]).start()

        def gather_wait(buf):
            pltpu.make_async_copy(
                table_hbm.at[idx_v.at[0]],
                rows[buf], gsem[buf]).wait()

        def out_start(p, buf):
            pltpu.make_async_copy(
                rows[buf], out_hbm.at[pl.ds((seq0 + p) * LP, LP)],
                osem[buf]).start()

        def out_wait(buf):
            pltpu.make_async_copy(
                rows[buf], out_hbm.at[pl.ds(0, LP)], osem[buf]).wait()

        gather_start(0, 0)
        gather_start(1, 1)

        def loop_body(pp, c):
            for buf in range(2):
                p = pp * 2 + buf
                gather_wait(buf)
                out_start(p, buf)
                out_wait(buf)
                gather_start(p + 2, buf)
            return c

        lax.fori_loop(0, SPW // 2 - 1, loop_body, 0)
        for buf in range(2):
            p = SPW - 2 + buf
            gather_wait(buf)
            out_start(p, buf)
            out_wait(buf)

    return k(ids3, table)


SB = 64


def _ln_body(x_ref, padd_ref, g_ref, bta_ref, o_ref):
    e = x_ref[:, :L, :] + padd_ref[...][None, :, :]
    mu = jnp.mean(e, axis=-1, keepdims=True)
    d = e - mu
    var = jnp.mean(d * d, axis=-1, keepdims=True)
    o_ref[...] = d * lax.rsqrt(var + EPS) * g_ref[...][None, :, :] \
        + bta_ref[...][None, :, :]


def _tc_add_ln(stag, padd, gamma2, beta2):
    return pl.pallas_call(
        _ln_body,
        grid=(B // SB,),
        in_specs=[
            pl.BlockSpec((SB, LP, D), lambda i: (i, 0, 0)),
            pl.BlockSpec((L, D), lambda i: (0, 0)),
            pl.BlockSpec((1, D), lambda i: (0, 0)),
            pl.BlockSpec((1, D), lambda i: (0, 0)),
        ],
        out_specs=pl.BlockSpec((SB, L, D), lambda i: (i, 0, 0)),
        out_shape=jax.ShapeDtypeStruct((B, L, D), jnp.float32),
        compiler_params=pltpu.CompilerParams(
            dimension_semantics=("arbitrary",),
        ),
    )(stag, padd, gamma2, beta2)


def kernel(input_ids, word_emb, pos_emb, type_emb, ln_gamma, ln_beta):
    ids32 = input_ids.astype(jnp.int32)
    ids_p = jnp.concatenate([ids32, ids32[:, :LP - L]], axis=1)
    ids3 = ids_p.reshape(NW, SPW, LP)
    stag = _sc_gather(ids3, word_emb).reshape(B, LP, D)
    padd = pos_emb[:L] + type_emb[0][None, :]
    return _tc_add_ln(stag, padd, ln_gamma.reshape(1, D),
                      ln_beta.reshape(1, D))


# speedup vs baseline: 1.3973x; 1.0001x over previous
import functools

import jax
import jax.numpy as jnp
from jax import lax
from jax.experimental import pallas as pl
from jax.experimental.pallas import tpu as pltpu
from jax.experimental.pallas import tpu_sc as plsc

B, L, D = 1024, 50, 768
N = B * L
EPS = 1e-12

NC, NS = 2, 16
NW = NC * NS
SPW = B // NW
LP = 56
NBUF = 3


def _sc_gather(ids3, table):
    mesh = plsc.VectorSubcoreMesh(core_axis_name="c", subcore_axis_name="s")

    @functools.partial(
        pl.kernel,
        mesh=mesh,
        out_type=jax.ShapeDtypeStruct((B * LP, D), jnp.float32),
        scratch_types=[
            pltpu.VMEM((SPW, LP), jnp.int32),
            pltpu.VMEM((LP, D), jnp.float32),
            pltpu.VMEM((LP, D), jnp.float32),
            pltpu.SemaphoreType.DMA,
            pltpu.SemaphoreType.DMA,
            pltpu.SemaphoreType.DMA,
            pltpu.SemaphoreType.DMA,
        ],
    )
    def k(ids_hbm, table_hbm, out_hbm, idx_v, rows_a, rows_b, gs_a, gs_b,
          os_a, os_b):
        wid = lax.axis_index("s") * NC + lax.axis_index("c")
        seq0 = wid * SPW

        pltpu.sync_copy(ids_hbm.at[wid], idx_v)

        rows = (rows_a, rows_b)
        gsem = (gs_a, gs_b)
        osem = (os_a, os_b)

        def gather_start(p, buf):
            pltpu.make_async_copy(
                table_hbm.at[idx_v.at[p]],
                rows[buf], gsem[buf]).start()

        def gather_wait(buf):
            pltpu.make_async_copy(
                table_hbm.at[idx_v.at[0]],
                rows[buf], gsem[buf]).wait()

        def out_start(p, buf):
            pltpu.make_async_copy(
                rows[buf], out_hbm.at[pl.ds((seq0 + p) * LP, LP)],
                osem[buf]).start()

        def out_wait(buf):
            pltpu.make_async_copy(
                rows[buf], out_hbm.at[pl.ds(0, LP)], osem[buf]).wait()

        gather_start(0, 0)
        gather_start(1, 1)

        def loop_body(pp, c):
            for buf in range(2):
                p = pp * 2 + buf
                gather_wait(buf)
                out_start(p, buf)
                out_wait(buf)
                gather_start(p + 2, buf)
            return c

        lax.fori_loop(0, SPW // 2 - 1, loop_body, 0)
        for buf in range(2):
            p = SPW - 2 + buf
            gather_wait(buf)
            out_start(p, buf)
            out_wait(buf)

    return k(ids3, table)


SB = 32


def _ln_body(x_ref, padd_ref, g_ref, bta_ref, o_ref):
    e = x_ref[:, :L, :] + padd_ref[...][None, :, :]
    mu = jnp.mean(e, axis=-1, keepdims=True)
    d = e - mu
    var = jnp.mean(d * d, axis=-1, keepdims=True)
    o_ref[...] = d * lax.rsqrt(var + EPS) * g_ref[...][None, :, :] \
        + bta_ref[...][None, :, :]


def _tc_add_ln(stag, padd, gamma2, beta2):
    return pl.pallas_call(
        _ln_body,
        grid=(B // SB,),
        in_specs=[
            pl.BlockSpec((SB, LP, D), lambda i: (i, 0, 0)),
            pl.BlockSpec((L, D), lambda i: (0, 0)),
            pl.BlockSpec((1, D), lambda i: (0, 0)),
            pl.BlockSpec((1, D), lambda i: (0, 0)),
        ],
        out_specs=pl.BlockSpec((SB, L, D), lambda i: (i, 0, 0)),
        out_shape=jax.ShapeDtypeStruct((B, L, D), jnp.float32),
        compiler_params=pltpu.CompilerParams(
            dimension_semantics=("arbitrary",),
        ),
    )(stag, padd, gamma2, beta2)


def kernel(input_ids, word_emb, pos_emb, type_emb, ln_gamma, ln_beta):
    ids32 = input_ids.astype(jnp.int32)
    ids_p = jnp.concatenate([ids32, ids32[:, :LP - L]], axis=1)
    ids3 = ids_p.reshape(NW, SPW, LP)
    stag = _sc_gather(ids3, word_emb).reshape(B, LP, D)
    padd = pos_emb[:L] + type_emb[0][None, :]
    return _tc_add_ln(stag, padd, ln_gamma.reshape(1, D),
                      ln_beta.reshape(1, D))
